# baseline (speedup 1.0000x reference)
import jax
import jax.numpy as jnp
from jax import lax
from jax.experimental import pallas as pl
from jax.experimental.pallas import tpu as pltpu
from jax.experimental.pallas import tpu_sc as plsc

_N = 10000
_E = 320000
_G = 64
_NC = 2
_NS = 16
_NW = _NC * _NS
_CHUNK = 128
_ITERS = 78
_TOTCH = _NW * _ITERS
_NTAIL = (_E - _TOTCH * _CHUNK) // _CHUNK
_NP = _N + 8
_RPT = 624
_TAIL = _NS * _RPT
_CW = 16

_f32 = jnp.float32


def _make_agg(D, with_cnt, dt):
  lanes = 16 * 4 // jnp.dtype(dt).itemsize
  mesh = plsc.VectorSubcoreMesh(
      core_axis_name="c", subcore_axis_name="s",
      num_cores=_NC, num_subcores=_NS)
  out_type = [jax.ShapeDtypeStruct((_NC, _N, D), dt)]
  scratch = [
      pltpu.VMEM((_ITERS, _CHUNK), jnp.int32),
      pltpu.VMEM((_ITERS, _CHUNK), jnp.int32),
      pltpu.VMEM((_CHUNK,), jnp.int32),
      pltpu.VMEM((_CHUNK,), jnp.int32),
      pltpu.VMEM((_CHUNK, D), dt),
      pltpu.VMEM((_CHUNK, D), dt),
      pltpu.VMEM_SHARED((_NP, D), dt),
      pltpu.VMEM_SHARED((_NP, D), dt),
      pltpu.SemaphoreType.DMA,
      pltpu.SemaphoreType.DMA,
  ]
  if with_cnt:
    out_type.append(jax.ShapeDtypeStruct((_NC, _N, _CW), jnp.bfloat16))
    scratch += [
        pltpu.VMEM((_CHUNK, _CW), jnp.bfloat16),
        pltpu.VMEM_SHARED((_NP, _CW), jnp.bfloat16),
    ]

  def body(*refs):
    if with_cnt:
      (p_hbm, srcs_hbm, dsts_hbm, tsrcs_hbm, tdsts_hbm, out_hbm, cnt_hbm,
       src_all, dst_all, tsrc_v, tdst_v, rows0, rows1, acc_sh, tab_sh,
       sem0, sem1, ones_v, cnt_sh) = refs
    else:
      (p_hbm, srcs_hbm, dsts_hbm, tsrcs_hbm, tdsts_hbm, out_hbm,
       src_all, dst_all, tsrc_v, tdst_v, rows0, rows1, acc_sh, tab_sh,
       sem0, sem1) = refs
    cid = lax.axis_index("c")
    sid = lax.axis_index("s")
    w = cid * _NS + sid

    zero16 = jnp.zeros((16,), _f32)
    zerov = jnp.zeros((lanes,), dt)
    for r in range(_CHUNK):
      for c in range(D // lanes):
        rows0[r, pl.ds(c * lanes, lanes)] = zerov
    base_r = pl.multiple_of(sid * _RPT, 8)
    nfull, rem = divmod(_RPT, _CHUNK)
    for j in range(nfull):
      pltpu.sync_copy(rows0, acc_sh.at[pl.ds(base_r + j * _CHUNK, _CHUNK)])
    if rem:
      pltpu.sync_copy(rows0.at[pl.ds(0, rem)],
                      acc_sh.at[pl.ds(base_r + nfull * _CHUNK, rem)])

    @pl.when(sid == 0)
    def _():
      pltpu.sync_copy(rows0.at[pl.ds(0, _NP - _TAIL)],
                      acc_sh.at[pl.ds(_TAIL, _NP - _TAIL)])

    pltpu.sync_copy(p_hbm.at[pl.ds(base_r, _RPT)],
                    tab_sh.at[pl.ds(base_r, _RPT)])

    @pl.when(sid == 0)
    def _():
      pltpu.sync_copy(p_hbm.at[pl.ds(_TAIL, _N - _TAIL)],
                      tab_sh.at[pl.ds(_TAIL, _N - _TAIL)])

    if with_cnt:
      zero32b = jnp.zeros((32,), jnp.bfloat16)
      for r in range(_CHUNK // 2):
        ones_v[pl.ds(2 * r, 2), :] = jnp.reshape(zero32b, (2, 16))
      for j in range(nfull):
        pltpu.sync_copy(ones_v, cnt_sh.at[pl.ds(base_r + j * _CHUNK, _CHUNK)])
      if rem:
        pltpu.sync_copy(ones_v.at[pl.ds(0, rem)],
                        cnt_sh.at[pl.ds(base_r + nfull * _CHUNK, rem)])

      @pl.when(sid == 0)
      def _():
        pltpu.sync_copy(ones_v.at[pl.ds(0, _NP - _TAIL)],
                        cnt_sh.at[pl.ds(_TAIL, _NP - _TAIL)])

      one32b = jnp.ones((32,), jnp.bfloat16)
      for r in range(_CHUNK // 2):
        ones_v[pl.ds(2 * r, 2), :] = jnp.reshape(one32b, (2, 16))
    plsc.subcore_barrier()

    pltpu.sync_copy(srcs_hbm.at[pl.ds(w * _ITERS, _ITERS)], src_all)
    pltpu.sync_copy(dsts_hbm.at[pl.ds(w * _ITERS, _ITERS)], dst_all)

    def gather(i, buf, sem):
      return pltpu.async_copy(tab_sh.at[src_all.at[i]], buf, sem)

    def scatter(i, buf):
      pltpu.sync_copy(buf, acc_sh.at[dst_all.at[i]], add=True)
      if with_cnt:
        pltpu.sync_copy(ones_v, cnt_sh.at[dst_all.at[i]], add=True)

    gather(0, rows0, sem0)

    def pair(j, carry):
      i0 = 2 * j
      pltpu.make_async_copy(tab_sh.at[src_all.at[i0]], rows0, sem0).wait()
      gather(i0 + 1, rows1, sem1)
      scatter(i0, rows0)
      pltpu.make_async_copy(tab_sh.at[src_all.at[i0 + 1]], rows1, sem1).wait()
      gather(i0 + 2, rows0, sem0)
      scatter(i0 + 1, rows1)
      return carry

    lax.fori_loop(0, (_ITERS - 2) // 2, pair, 0)
    pltpu.make_async_copy(tab_sh.at[src_all.at[_ITERS - 2]], rows0, sem0).wait()
    gather(_ITERS - 1, rows1, sem1)
    scatter(_ITERS - 2, rows0)
    pltpu.make_async_copy(tab_sh.at[src_all.at[_ITERS - 1]], rows1, sem1).wait()
    scatter(_ITERS - 1, rows1)

    @pl.when(w < _NTAIL)
    def _():
      pltpu.sync_copy(tsrcs_hbm.at[w], tsrc_v)
      pltpu.sync_copy(tdsts_hbm.at[w], tdst_v)
      pltpu.async_copy(tab_sh.at[tsrc_v], rows0, sem0).wait()
      pltpu.sync_copy(rows0, acc_sh.at[tdst_v], add=True)
      if with_cnt:
        pltpu.sync_copy(ones_v, cnt_sh.at[tdst_v], add=True)
    plsc.subcore_barrier()

    pltpu.sync_copy(acc_sh.at[pl.ds(base_r, _RPT)],
                    out_hbm.at[cid, pl.ds(base_r, _RPT)])
    if with_cnt:
      pltpu.sync_copy(cnt_sh.at[pl.ds(base_r, _RPT)],
                      cnt_hbm.at[cid, pl.ds(base_r, _RPT)])

    @pl.when(sid == 0)
    def _():
      pltpu.sync_copy(acc_sh.at[pl.ds(_TAIL, _N - _TAIL)],
                      out_hbm.at[cid, pl.ds(_TAIL, _N - _TAIL)])
      if with_cnt:
        pltpu.sync_copy(cnt_sh.at[pl.ds(_TAIL, _N - _TAIL)],
                        cnt_hbm.at[cid, pl.ds(_TAIL, _N - _TAIL)])

  return pl.kernel(body, out_type=out_type, mesh=mesh, scratch_types=scratch,
                   compiler_params=pltpu.CompilerParams(
                       use_tc_tiling_on_sc=False))


_bf = jnp.bfloat16
_agg64_cnt = _make_agg(64, with_cnt=True, dt=_bf)
_agg32 = _make_agg(32, with_cnt=False, dt=_bf)


def _proj1_body(x_ref, wn_ref, ws_ref, b_ref, p_ref, s_ref):
  x = x_ref[...]
  p_ref[...] = jnp.dot(
      x, wn_ref[...], preferred_element_type=_f32).astype(_bf)
  s_ref[...] = (jnp.dot(x, ws_ref[...], preferred_element_type=_f32)
                + b_ref[...][None, :])


_proj1 = pl.pallas_call(
    _proj1_body,
    out_shape=[jax.ShapeDtypeStruct((_N, 64), _bf),
               jax.ShapeDtypeStruct((_N, 64), _f32)])


def _mid1_body(aggp_ref, cntp_ref, s1_ref, g1_ref, be1_ref,
               w2n_ref, w2s_ref, b2_ref, p2_ref, s2_ref, inv_ref):
  cnt = (cntp_ref[0, :, 0:1].astype(_f32)
         + cntp_ref[1, :, 0:1].astype(_f32))
  inv = 1.0 / jnp.maximum(cnt, 1.0)
  inv_ref[...] = inv
  agg = aggp_ref[0].astype(_f32) + aggp_ref[1].astype(_f32)
  z = agg * inv + s1_ref[...]
  mu = jnp.mean(z, axis=0, keepdims=True)
  var = jnp.mean((z - mu) ** 2, axis=0, keepdims=True)
  h = (z - mu) / jnp.sqrt(var + 1e-5) * g1_ref[...][None, :] \
      + be1_ref[...][None, :]
  h = jnp.maximum(h, 0.0)
  p2_ref[...] = jnp.dot(
      h, w2n_ref[...], preferred_element_type=_f32).astype(_bf)
  s2_ref[...] = (jnp.dot(h, w2s_ref[...], preferred_element_type=_f32)
                 + b2_ref[...][None, :])


_mid1 = pl.pallas_call(
    _mid1_body,
    out_shape=[jax.ShapeDtypeStruct((_N, 32), _bf),
               jax.ShapeDtypeStruct((_N, 32), _f32),
               jax.ShapeDtypeStruct((_N, 1), _f32)])


def _mid2_body(aggp_ref, inv_ref, s2_ref, w3n_ref, w3s_ref, b3_ref,
               p3_ref, s3_ref):
  inv = inv_ref[...]
  agg = aggp_ref[0].astype(_f32) + aggp_ref[1].astype(_f32)
  h = jnp.maximum(agg * inv + s2_ref[...], 0.0)
  p3_ref[...] = jnp.dot(
      h, w3n_ref[...], preferred_element_type=_f32).astype(_bf)
  s3_ref[...] = (jnp.dot(h, w3s_ref[...], preferred_element_type=_f32)
                 + b3_ref[...][None, :])


_mid2 = pl.pallas_call(
    _mid2_body,
    out_shape=[jax.ShapeDtypeStruct((_N, 32), _bf),
               jax.ShapeDtypeStruct((_N, 20), _f32)])


def _final_body(aggp_ref, inv_ref, s3_ref, g2_ref, be2_ref, batch_ref,
                wl_ref, bl_ref, out_ref):
  inv = inv_ref[...]
  agg = (aggp_ref[0][:, :20].astype(_f32)
         + aggp_ref[1][:, :20].astype(_f32))
  z = agg * inv + s3_ref[...]
  mu = jnp.mean(z, axis=0, keepdims=True)
  var = jnp.mean((z - mu) ** 2, axis=0, keepdims=True)
  h = (z - mu) / jnp.sqrt(var + 1e-5) * g2_ref[...][None, :] \
      + be2_ref[...][None, :]
  b = batch_ref[...]
  oh_t = (b[None, :] == lax.broadcasted_iota(jnp.int32, (_G, _N), 0))
  oh_t = oh_t.astype(_f32)
  pooled_sum = jnp.dot(oh_t, h, preferred_element_type=_f32)
  cg = jnp.sum(oh_t, axis=1, keepdims=True)
  pooled = pooled_sum / jnp.maximum(cg, 1.0)
  logits = (jnp.dot(pooled, wl_ref[...], preferred_element_type=_f32)
            + bl_ref[...][None, :])
  m = jnp.max(logits, axis=-1, keepdims=True)
  e = jnp.exp(logits - m)
  out_ref[...] = e / jnp.sum(e, axis=-1, keepdims=True)


_final = pl.pallas_call(
    _final_body,
    out_shape=jax.ShapeDtypeStruct((_G, 11), _f32))


def kernel(x, edge_index, batch, z, W1s, W1n, b1, g1, be1, W2s, W2n, b2,
           W3s, W3n, b3, g2, be2, Wl, bl):
  nfull = _TOTCH * _CHUNK
  srcs = edge_index[0][:nfull].reshape(_TOTCH, _CHUNK)
  dsts = edge_index[1][:nfull].reshape(_TOTCH, _CHUNK)
  tsrcs = edge_index[0][nfull:].reshape(_NTAIL, _CHUNK)
  tdsts = edge_index[1][nfull:].reshape(_NTAIL, _CHUNK)
  p1, s1 = _proj1(x, W1n, W1s, b1)
  agg1p, cntp = _agg64_cnt(p1, srcs, dsts, tsrcs, tdsts)
  p2, s2, inv = _mid1(agg1p, cntp, s1, g1, be1, W2n, W2s, b2)
  agg2p, = _agg32(p2, srcs, dsts, tsrcs, tdsts)
  w3n_pad = jnp.concatenate([W3n, jnp.zeros((32, 12), _f32)], axis=1)
  p3, s3 = _mid2(agg2p, inv, s2, w3n_pad, W3s, b3)
  agg3p, = _agg32(p3, srcs, dsts, tsrcs, tdsts)
  return _final(agg3p, inv, s3, g2, be2, batch, Wl, bl)


# speedup vs baseline: 1.0125x; 1.0125x over previous
import jax
import jax.numpy as jnp
from jax import lax
from jax.experimental import pallas as pl
from jax.experimental.pallas import tpu as pltpu
from jax.experimental.pallas import tpu_sc as plsc

_N = 10000
_E = 320000
_G = 64
_NC = 2
_NS = 16
_NW = _NC * _NS
_CHUNK = 128
_ITERS = 79
_EW = _ITERS * _CHUNK
_EP = _NW * _EW
_NP = _N + 8
_RPT = 624
_TAIL = _NS * _RPT
_CW = 16

_f32 = jnp.float32


def _make_agg(D, with_cnt, dt):
  lanes = 16 * 4 // jnp.dtype(dt).itemsize
  mesh = plsc.VectorSubcoreMesh(
      core_axis_name="c", subcore_axis_name="s",
      num_cores=_NC, num_subcores=_NS)
  out_type = [jax.ShapeDtypeStruct((_NC, _N, D), dt)]
  scratch = [
      pltpu.VMEM((_ITERS, _CHUNK), jnp.int32),
      pltpu.VMEM((_ITERS, _CHUNK), jnp.int32),
      pltpu.VMEM((_CHUNK, D), dt),
      pltpu.VMEM((_CHUNK, D), dt),
      pltpu.VMEM_SHARED((_NP, D), dt),
      pltpu.VMEM_SHARED((_NP, D), dt),
      pltpu.SemaphoreType.DMA,
      pltpu.SemaphoreType.DMA,
  ]
  if with_cnt:
    out_type.append(jax.ShapeDtypeStruct((_NC, _N, _CW), jnp.bfloat16))
    scratch += [
        pltpu.VMEM((_CHUNK, _CW), jnp.bfloat16),
        pltpu.VMEM_SHARED((_NP, _CW), jnp.bfloat16),
    ]

  def body(*refs):
    if with_cnt:
      (p_hbm, srcs_hbm, dsts_hbm, out_hbm, cnt_hbm,
       src_all, dst_all, rows0, rows1, acc_sh, tab_sh, sem0, sem1,
       ones_v, cnt_sh) = refs
    else:
      (p_hbm, srcs_hbm, dsts_hbm, out_hbm,
       src_all, dst_all, rows0, rows1, acc_sh, tab_sh, sem0, sem1) = refs
    cid = lax.axis_index("c")
    sid = lax.axis_index("s")
    w = cid * _NS + sid

    zero16 = jnp.zeros((16,), _f32)
    zerov = jnp.zeros((lanes,), dt)
    for r in range(_CHUNK):
      for c in range(D // lanes):
        rows0[r, pl.ds(c * lanes, lanes)] = zerov
    base_r = pl.multiple_of(sid * _RPT, 8)
    nfull, rem = divmod(_RPT, _CHUNK)
    for j in range(nfull):
      pltpu.sync_copy(rows0, acc_sh.at[pl.ds(base_r + j * _CHUNK, _CHUNK)])
    if rem:
      pltpu.sync_copy(rows0.at[pl.ds(0, rem)],
                      acc_sh.at[pl.ds(base_r + nfull * _CHUNK, rem)])

    @pl.when(sid == 0)
    def _():
      pltpu.sync_copy(rows0.at[pl.ds(0, _NP - _TAIL)],
                      acc_sh.at[pl.ds(_TAIL, _NP - _TAIL)])

    pltpu.sync_copy(p_hbm.at[pl.ds(base_r, _RPT)],
                    tab_sh.at[pl.ds(base_r, _RPT)])

    @pl.when(sid == 0)
    def _():
      pltpu.sync_copy(p_hbm.at[pl.ds(_TAIL, _NP - _TAIL)],
                      tab_sh.at[pl.ds(_TAIL, _NP - _TAIL)])

    if with_cnt:
      zero32b = jnp.zeros((32,), jnp.bfloat16)
      for r in range(_CHUNK // 2):
        ones_v[pl.ds(2 * r, 2), :] = jnp.reshape(zero32b, (2, 16))
      for j in range(nfull):
        pltpu.sync_copy(ones_v, cnt_sh.at[pl.ds(base_r + j * _CHUNK, _CHUNK)])
      if rem:
        pltpu.sync_copy(ones_v.at[pl.ds(0, rem)],
                        cnt_sh.at[pl.ds(base_r + nfull * _CHUNK, rem)])

      @pl.when(sid == 0)
      def _():
        pltpu.sync_copy(ones_v.at[pl.ds(0, _NP - _TAIL)],
                        cnt_sh.at[pl.ds(_TAIL, _NP - _TAIL)])

      one32b = jnp.ones((32,), jnp.bfloat16)
      for r in range(_CHUNK // 2):
        ones_v[pl.ds(2 * r, 2), :] = jnp.reshape(one32b, (2, 16))
    plsc.subcore_barrier()

    pltpu.sync_copy(srcs_hbm.at[w], src_all)
    pltpu.sync_copy(dsts_hbm.at[w], dst_all)

    def gather(i, buf, sem):
      return pltpu.async_copy(tab_sh.at[src_all.at[i]], buf, sem)

    def scatter(i, buf):
      pltpu.sync_copy(buf, acc_sh.at[dst_all.at[i]], add=True)
      if with_cnt:
        pltpu.sync_copy(ones_v, cnt_sh.at[dst_all.at[i]], add=True)

    gather(0, rows0, sem0)

    def pair(j, carry):
      i0 = 2 * j
      pltpu.make_async_copy(tab_sh.at[src_all.at[i0]], rows0, sem0).wait()
      gather(i0 + 1, rows1, sem1)
      scatter(i0, rows0)
      pltpu.make_async_copy(tab_sh.at[src_all.at[i0 + 1]], rows1, sem1).wait()
      gather(i0 + 2, rows0, sem0)
      scatter(i0 + 1, rows1)
      return carry

    lax.fori_loop(0, (_ITERS - 1) // 2, pair, 0)
    pltpu.make_async_copy(tab_sh.at[src_all.at[_ITERS - 1]], rows0, sem0).wait()
    scatter(_ITERS - 1, rows0)
    plsc.subcore_barrier()

    pltpu.sync_copy(acc_sh.at[pl.ds(base_r, _RPT)],
                    out_hbm.at[cid, pl.ds(base_r, _RPT)])
    if with_cnt:
      pltpu.sync_copy(cnt_sh.at[pl.ds(base_r, _RPT)],
                      cnt_hbm.at[cid, pl.ds(base_r, _RPT)])

    @pl.when(sid == 0)
    def _():
      pltpu.sync_copy(acc_sh.at[pl.ds(_TAIL, _N - _TAIL)],
                      out_hbm.at[cid, pl.ds(_TAIL, _N - _TAIL)])
      if with_cnt:
        pltpu.sync_copy(cnt_sh.at[pl.ds(_TAIL, _N - _TAIL)],
                        cnt_hbm.at[cid, pl.ds(_TAIL, _N - _TAIL)])

  return pl.kernel(body, out_type=out_type, mesh=mesh, scratch_types=scratch,
                   compiler_params=pltpu.CompilerParams(
                       use_tc_tiling_on_sc=False))


_bf = jnp.bfloat16
_agg64_cnt = _make_agg(64, with_cnt=True, dt=_bf)
_agg32 = _make_agg(32, with_cnt=False, dt=_bf)


def _proj1_body(x_ref, wn_ref, ws_ref, b_ref, p_ref, s_ref):
  x = x_ref[...]
  p_ref[pl.ds(0, _N), :] = jnp.dot(
      x, wn_ref[...], preferred_element_type=_f32).astype(_bf)
  s_ref[...] = (jnp.dot(x, ws_ref[...], preferred_element_type=_f32)
                + b_ref[...][None, :])


_proj1 = pl.pallas_call(
    _proj1_body,
    out_shape=[jax.ShapeDtypeStruct((_NP, 64), _bf),
               jax.ShapeDtypeStruct((_N, 64), _f32)])


def _mid1_body(aggp_ref, cntp_ref, s1_ref, g1_ref, be1_ref,
               w2n_ref, w2s_ref, b2_ref, p2_ref, s2_ref, inv_ref):
  cnt = (cntp_ref[0, :, 0:1].astype(_f32)
         + cntp_ref[1, :, 0:1].astype(_f32))
  inv = 1.0 / jnp.maximum(cnt, 1.0)
  inv_ref[...] = inv
  agg = aggp_ref[0].astype(_f32) + aggp_ref[1].astype(_f32)
  z = agg * inv + s1_ref[...]
  mu = jnp.mean(z, axis=0, keepdims=True)
  var = jnp.mean((z - mu) ** 2, axis=0, keepdims=True)
  h = (z - mu) / jnp.sqrt(var + 1e-5) * g1_ref[...][None, :] \
      + be1_ref[...][None, :]
  h = jnp.maximum(h, 0.0)
  p2_ref[pl.ds(0, _N), :] = jnp.dot(
      h, w2n_ref[...], preferred_element_type=_f32).astype(_bf)
  s2_ref[...] = (jnp.dot(h, w2s_ref[...], preferred_element_type=_f32)
                 + b2_ref[...][None, :])


_mid1 = pl.pallas_call(
    _mid1_body,
    out_shape=[jax.ShapeDtypeStruct((_NP, 32), _bf),
               jax.ShapeDtypeStruct((_N, 32), _f32),
               jax.ShapeDtypeStruct((_N, 1), _f32)])


def _mid2_body(aggp_ref, inv_ref, s2_ref, w3n_ref, w3s_ref, b3_ref,
               p3_ref, s3_ref):
  inv = inv_ref[...]
  agg = aggp_ref[0].astype(_f32) + aggp_ref[1].astype(_f32)
  h = jnp.maximum(agg * inv + s2_ref[...], 0.0)
  p3_ref[pl.ds(0, _N), :] = jnp.dot(
      h, w3n_ref[...], preferred_element_type=_f32).astype(_bf)
  s3_ref[...] = (jnp.dot(h, w3s_ref[...], preferred_element_type=_f32)
                 + b3_ref[...][None, :])


_mid2 = pl.pallas_call(
    _mid2_body,
    out_shape=[jax.ShapeDtypeStruct((_NP, 32), _bf),
               jax.ShapeDtypeStruct((_N, 20), _f32)])


def _final_body(aggp_ref, inv_ref, s3_ref, g2_ref, be2_ref, batch_ref,
                wl_ref, bl_ref, out_ref):
  inv = inv_ref[...]
  agg = (aggp_ref[0][:, :20].astype(_f32)
         + aggp_ref[1][:, :20].astype(_f32))
  z = agg * inv + s3_ref[...]
  mu = jnp.mean(z, axis=0, keepdims=True)
  var = jnp.mean((z - mu) ** 2, axis=0, keepdims=True)
  h = (z - mu) / jnp.sqrt(var + 1e-5) * g2_ref[...][None, :] \
      + be2_ref[...][None, :]
  b = batch_ref[...]
  oh_t = (b[None, :] == lax.broadcasted_iota(jnp.int32, (_G, _N), 0))
  oh_t = oh_t.astype(_f32)
  pooled_sum = jnp.dot(oh_t, h, preferred_element_type=_f32)
  cg = jnp.sum(oh_t, axis=1, keepdims=True)
  pooled = pooled_sum / jnp.maximum(cg, 1.0)
  logits = (jnp.dot(pooled, wl_ref[...], preferred_element_type=_f32)
            + bl_ref[...][None, :])
  m = jnp.max(logits, axis=-1, keepdims=True)
  e = jnp.exp(logits - m)
  out_ref[...] = e / jnp.sum(e, axis=-1, keepdims=True)


_final = pl.pallas_call(
    _final_body,
    out_shape=jax.ShapeDtypeStruct((_G, 11), _f32))


def kernel(x, edge_index, batch, z, W1s, W1n, b1, g1, be1, W2s, W2n, b2,
           W3s, W3n, b3, g2, be2, Wl, bl):
  pad = jnp.full((_EP - _E,), _N, jnp.int32)
  srcs = jnp.concatenate([edge_index[0], pad]).reshape(_NW, _ITERS, _CHUNK)
  dsts = jnp.concatenate([edge_index[1], pad]).reshape(_NW, _ITERS, _CHUNK)
  p1, s1 = _proj1(x, W1n, W1s, b1)
  agg1p, cntp = _agg64_cnt(p1, srcs, dsts)
  p2, s2, inv = _mid1(agg1p, cntp, s1, g1, be1, W2n, W2s, b2)
  agg2p, = _agg32(p2, srcs, dsts)
  w3n_pad = jnp.concatenate([W3n, jnp.zeros((32, 12), _f32)], axis=1)
  p3, s3 = _mid2(agg2p, inv, s2, w3n_pad, W3s, b3)
  agg3p, = _agg32(p3, srcs, dsts)
  return _final(agg3p, inv, s3, g2, be2, batch, Wl, bl)
